# Initial kernel scaffold; baseline (speedup 1.0000x reference)
import jax
import jax.numpy as jnp
from jax.experimental import pallas as pl


def kernel(x, Wq, bq, Wk, bk, Wv, bv, Wp, bp):
    raise NotImplementedError("write your pallas kernel here")


# speedup vs baseline: 10.4521x; 10.4521x over previous
import math

import jax
import jax.numpy as jnp
from jax.experimental import pallas as pl
from jax.experimental.pallas import tpu as pltpu

_HEADS = 12
_LOGK_FACTOR = 5
_TB = 512
_TM = 512
_TL = 512


def _mm_bias_kernel(x_ref, w_ref, b_ref, o_ref):
    xb = x_ref[...].astype(jnp.bfloat16)
    wb = w_ref[...].astype(jnp.bfloat16)
    o_ref[...] = (
        jnp.dot(xb, wb, preferred_element_type=jnp.float32) + b_ref[...]
    )


def _matmul_bias(x2d, w, bias):
    m, kdim = x2d.shape
    n = w.shape[1]
    grid = (m // _TM,)
    return pl.pallas_call(
        _mm_bias_kernel,
        grid=grid,
        in_specs=[
            pl.BlockSpec((_TM, kdim), lambda i: (i, 0)),
            pl.BlockSpec((kdim, n), lambda i: (0, 0)),
            pl.BlockSpec((1, n), lambda i: (0, 0)),
        ],
        out_specs=pl.BlockSpec((_TM, n), lambda i: (i, 0)),
        out_shape=jax.ShapeDtypeStruct((m, n), jnp.float32),
    )(x2d, w, bias)


def _corr_kernel(q_ref, k_ref, o_ref, *, nt, tb):
    d = pl.program_id(1)
    i = pl.program_id(2)

    @pl.when((d == 0) & (i == 0))
    def _():
        o_ref[...] = jnp.zeros_like(o_ref)

    s = jax.lax.dot_general(
        q_ref[0], k_ref[0],
        (((1,), (1,)), ((), ())),
        preferred_element_type=jnp.float32,
        precision=jax.lax.Precision.HIGHEST,
    )

    a_i = jax.lax.broadcasted_iota(jnp.int32, (tb, tb), 0)
    c_i = jax.lax.broadcasted_iota(jnp.int32, (tb, tb), 1)

    rolled = s
    kbit = 0
    while (1 << kbit) < tb:
        sh = 1 << kbit
        bit = (a_i >> kbit) & 1
        rolled = jnp.where(bit == 1, pltpu.roll(rolled, tb - sh, axis=1), rolled)
        kbit += 1

    mask1 = (a_i + c_i) < tb
    colsum1 = jnp.sum(jnp.where(mask1, rolled, 0.0), axis=0, keepdims=True)
    colsum2 = jnp.sum(jnp.where(mask1, 0.0, rolled), axis=0, keepdims=True)

    jmat = (a_i + c_i == tb - 1).astype(jnp.float32)
    rev1 = jnp.dot(colsum1, jmat, preferred_element_type=jnp.float32,
                   precision=jax.lax.Precision.HIGHEST)
    rev2 = jnp.dot(colsum2, jmat, preferred_element_type=jnp.float32,
                   precision=jax.lax.Precision.HIGHEST)
    contrib = jnp.concatenate([rev1, rev2], axis=1)

    cur = o_ref[0, pl.ds(d, 1), :]
    o_ref[0, pl.ds(d, 1), :] = cur + contrib


def _corr_partials(q, k):
    bsz, l, c = q.shape
    nt = l // _TB
    kern = lambda qr, kr, orf: _corr_kernel(qr, kr, orf, nt=nt, tb=_TB)
    return pl.pallas_call(
        kern,
        grid=(bsz, nt, nt),
        in_specs=[
            pl.BlockSpec((1, _TB, c), lambda b, d, i: (b, i, 0)),
            pl.BlockSpec((1, _TB, c), lambda b, d, i: (b, (i - d) % nt, 0)),
        ],
        out_specs=pl.BlockSpec((1, nt, 2 * _TB), lambda b, d, i: (b, 0, 0)),
        out_shape=jax.ShapeDtypeStruct((bsz, nt, 2 * _TB), jnp.float32),
    )(q, k)


def _topk_kernel(w_ref, wout_ref, dout_ref, *, nt, tb, l, ktop, cdim):
    w = w_ref[0]
    m_a = jnp.concatenate([w[1:, :tb], w[:1, :tb]], axis=0)
    m0 = (m_a + w[:, tb:]) * (1.0 / cdim)
    dd = jax.lax.broadcasted_iota(jnp.int32, (nt, tb), 0)
    ss = jax.lax.broadcasted_iota(jnp.int32, (nt, tb), 1)
    tau = (dd * tb + ss + 1) % l

    lane = jax.lax.broadcasted_iota(jnp.int32, (1, 128), 1)
    neg_inf = jnp.float32(-jnp.inf)

    def body(it, carry):
        m_cur, vals, idxs = carry
        cur = jnp.max(m_cur)
        idx = jnp.min(jnp.where(m_cur == cur, tau, l))
        vals = jnp.where(lane == it, cur, vals)
        idxs = jnp.where(lane == it, idx, idxs)
        m_cur = jnp.where((m_cur == cur) & (tau == idx), neg_inf, m_cur)
        return m_cur, vals, idxs

    vals0 = jnp.full((1, 128), neg_inf, jnp.float32)
    idxs0 = jnp.zeros((1, 128), jnp.int32)
    _, vals, idxs = jax.lax.fori_loop(0, ktop, body, (m0, vals0, idxs0))

    active = lane < ktop
    mx = jnp.max(vals)
    e = jnp.where(active, jnp.exp(vals - mx), 0.0)
    wout_ref[0] = e / jnp.sum(e)
    dout_ref[0] = idxs


def _topk_softmax(w_all, l, cdim, ktop):
    bsz, nt, two_tb = w_all.shape
    tb = two_tb // 2
    kern = lambda wr, wo, do: _topk_kernel(
        wr, wo, do, nt=nt, tb=tb, l=l, ktop=ktop, cdim=cdim
    )
    return pl.pallas_call(
        kern,
        grid=(bsz,),
        in_specs=[pl.BlockSpec((1, nt, two_tb), lambda b: (b, 0, 0))],
        out_specs=[
            pl.BlockSpec((1, 1, 128), lambda b: (b, 0, 0)),
            pl.BlockSpec((1, 1, 128), lambda b: (b, 0, 0)),
        ],
        out_shape=[
            jax.ShapeDtypeStruct((bsz, 1, 128), jnp.float32),
            jax.ShapeDtypeStruct((bsz, 1, 128), jnp.int32),
        ],
    )(w_all)


def _agg_kernel(dhi_ref, dlo_ref, w_ref, vv_ref, o_ref, *, ktop, tl, cdim):
    b = pl.program_id(0)
    lt = pl.program_id(1)
    l0 = lt * tl
    win = tl + 8

    def body(it, acc):
        dhi = dhi_ref[b, it]
        dlo = dlo_ref[b, it]
        wgt = w_ref[b, it]
        base = pl.multiple_of(l0 + dhi * 8, 8)
        chunk = vv_ref[0, pl.ds(base, win), :]
        rolled = pltpu.roll(chunk, (win - dlo) % win, axis=0)
        return acc + wgt * rolled[:tl, :]

    acc0 = jnp.zeros((tl, cdim), jnp.float32)
    o_ref[0] = jax.lax.fori_loop(0, ktop, body, acc0)


def _delay_aggregate(vv, dhi, dlo, weights, ktop):
    bsz, two_l, cdim = vv.shape
    l = two_l // 2
    nl = l // _TL
    kern = lambda hr, lr, wr, vr, orf: _agg_kernel(
        hr, lr, wr, vr, orf, ktop=ktop, tl=_TL, cdim=cdim
    )
    return pl.pallas_call(
        kern,
        grid=(bsz, nl),
        in_specs=[
            pl.BlockSpec(memory_space=pltpu.SMEM),
            pl.BlockSpec(memory_space=pltpu.SMEM),
            pl.BlockSpec(memory_space=pltpu.SMEM),
            pl.BlockSpec((1, two_l, cdim), lambda b, t: (b, 0, 0)),
        ],
        out_specs=pl.BlockSpec((1, _TL, cdim), lambda b, t: (b, t, 0)),
        out_shape=jax.ShapeDtypeStruct((bsz, l, cdim), jnp.float32),
    )(dhi, dlo, weights, vv)


def kernel(x, Wq, bq, Wk, bk, Wv, bv, Wp, bp):
    bsz, t, c = x.shape
    h = _HEADS
    e = c // h
    l = t
    ktop = int(_LOGK_FACTOR * math.log(l))

    x2 = x.reshape(bsz * t, c)
    w_all = jnp.concatenate([Wq.T, Wk.T, Wv.T], axis=1)
    b_all = jnp.concatenate([bq, bk, bv]).reshape(1, 3 * c)
    qkv = _matmul_bias(x2, w_all, b_all)
    q = qkv[:, :c].reshape(bsz, t, c)
    k = qkv[:, c:2 * c].reshape(bsz, t, c)
    v = qkv[:, 2 * c:].reshape(bsz, t, c)

    corr_parts = _corr_partials(q, k)
    weights, delays = _topk_softmax(corr_parts, l, c, ktop)
    weights = weights.reshape(bsz, 128)
    delays = delays.reshape(bsz, 128)

    vv = jnp.concatenate([v, v], axis=1)
    nat = _delay_aggregate(vv, delays // 8, delays % 8, weights, ktop)

    scr = nat.reshape(bsz, l, h, e).transpose(0, 2, 1, 3).reshape(bsz * t, c)
    out = _matmul_bias(scr, Wp.T, bp.reshape(1, c))
    return out.reshape(bsz, t, c)


# speedup vs baseline: 11.4089x; 1.0915x over previous
import math

import jax
import jax.numpy as jnp
from jax.experimental import pallas as pl
from jax.experimental.pallas import tpu as pltpu

_HEADS = 12
_LOGK_FACTOR = 5
_TB = 512
_TM = 512
_TL = 512


def _mm_bias_kernel(x_ref, w_ref, b_ref, o_ref):
    xb = x_ref[...].astype(jnp.bfloat16)
    wb = w_ref[...].astype(jnp.bfloat16)
    o_ref[...] = (
        jnp.dot(xb, wb, preferred_element_type=jnp.float32) + b_ref[...]
    )


def _matmul_bias(x2d, w, bias):
    m, kdim = x2d.shape
    n = w.shape[1]
    grid = (m // _TM,)
    return pl.pallas_call(
        _mm_bias_kernel,
        grid=grid,
        in_specs=[
            pl.BlockSpec((_TM, kdim), lambda i: (i, 0)),
            pl.BlockSpec((kdim, n), lambda i: (0, 0)),
            pl.BlockSpec((1, n), lambda i: (0, 0)),
        ],
        out_specs=pl.BlockSpec((_TM, n), lambda i: (i, 0)),
        out_shape=jax.ShapeDtypeStruct((m, n), jnp.float32),
    )(x2d, w, bias)


def _corr_kernel(q_ref, k_ref, o_ref, *, nt, tb):
    d = pl.program_id(1)
    i = pl.program_id(2)

    @pl.when((d == 0) & (i == 0))
    def _():
        o_ref[...] = jnp.zeros_like(o_ref)

    qf, kf = q_ref[0], k_ref[0]
    qh = qf.astype(jnp.bfloat16)
    ql = (qf - qh.astype(jnp.float32)).astype(jnp.bfloat16)
    kh = kf.astype(jnp.bfloat16)
    kl = (kf - kh.astype(jnp.float32)).astype(jnp.bfloat16)
    dims = (((1,), (1,)), ((), ()))
    dotg = lambda a, b: jax.lax.dot_general(
        a, b, dims, preferred_element_type=jnp.float32)
    s = dotg(qh, kh) + dotg(qh, kl) + dotg(ql, kh)

    a_i = jax.lax.broadcasted_iota(jnp.int32, (tb, tb), 0)
    c_i = jax.lax.broadcasted_iota(jnp.int32, (tb, tb), 1)

    rolled = s
    kbit = 0
    while (1 << kbit) < tb:
        sh = 1 << kbit
        bit = (a_i >> kbit) & 1
        rolled = jnp.where(bit == 1, pltpu.roll(rolled, tb - sh, axis=1), rolled)
        kbit += 1

    mask1 = (a_i + c_i) < tb
    colsum1 = jnp.sum(jnp.where(mask1, rolled, 0.0), axis=0, keepdims=True)
    colsum2 = jnp.sum(jnp.where(mask1, 0.0, rolled), axis=0, keepdims=True)

    jmat = (a_i + c_i == tb - 1).astype(jnp.float32)
    rev1 = jnp.dot(colsum1, jmat, preferred_element_type=jnp.float32,
                   precision=jax.lax.Precision.HIGHEST)
    rev2 = jnp.dot(colsum2, jmat, preferred_element_type=jnp.float32,
                   precision=jax.lax.Precision.HIGHEST)
    contrib = jnp.concatenate([rev1, rev2], axis=1)

    cur = o_ref[0, pl.ds(d, 1), :]
    o_ref[0, pl.ds(d, 1), :] = cur + contrib


def _corr_partials(q, k):
    bsz, l, c = q.shape
    nt = l // _TB
    kern = lambda qr, kr, orf: _corr_kernel(qr, kr, orf, nt=nt, tb=_TB)
    return pl.pallas_call(
        kern,
        grid=(bsz, nt, nt),
        in_specs=[
            pl.BlockSpec((1, _TB, c), lambda b, d, i: (b, i, 0)),
            pl.BlockSpec((1, _TB, c), lambda b, d, i: (b, (i - d) % nt, 0)),
        ],
        out_specs=pl.BlockSpec((1, nt, 2 * _TB), lambda b, d, i: (b, 0, 0)),
        out_shape=jax.ShapeDtypeStruct((bsz, nt, 2 * _TB), jnp.float32),
    )(q, k)


def _topk_kernel(w_ref, wout_ref, dout_ref, *, nt, tb, l, ktop, cdim):
    w = w_ref[0]
    m_a = jnp.concatenate([w[1:, :tb], w[:1, :tb]], axis=0)
    m0 = (m_a + w[:, tb:]) * (1.0 / cdim)
    dd = jax.lax.broadcasted_iota(jnp.int32, (nt, tb), 0)
    ss = jax.lax.broadcasted_iota(jnp.int32, (nt, tb), 1)
    tau = (dd * tb + ss + 1) % l

    lane = jax.lax.broadcasted_iota(jnp.int32, (1, 128), 1)
    neg_inf = jnp.float32(-jnp.inf)

    def body(it, carry):
        m_cur, vals, idxs = carry
        cur = jnp.max(m_cur)
        idx = jnp.min(jnp.where(m_cur == cur, tau, l))
        vals = jnp.where(lane == it, cur, vals)
        idxs = jnp.where(lane == it, idx, idxs)
        m_cur = jnp.where((m_cur == cur) & (tau == idx), neg_inf, m_cur)
        return m_cur, vals, idxs

    vals0 = jnp.full((1, 128), neg_inf, jnp.float32)
    idxs0 = jnp.zeros((1, 128), jnp.int32)
    _, vals, idxs = jax.lax.fori_loop(0, ktop, body, (m0, vals0, idxs0))

    active = lane < ktop
    mx = jnp.max(vals)
    e = jnp.where(active, jnp.exp(vals - mx), 0.0)
    wout_ref[0] = e / jnp.sum(e)
    dout_ref[0] = idxs


def _topk_softmax(w_all, l, cdim, ktop):
    bsz, nt, two_tb = w_all.shape
    tb = two_tb // 2
    kern = lambda wr, wo, do: _topk_kernel(
        wr, wo, do, nt=nt, tb=tb, l=l, ktop=ktop, cdim=cdim
    )
    return pl.pallas_call(
        kern,
        grid=(bsz,),
        in_specs=[pl.BlockSpec((1, nt, two_tb), lambda b: (b, 0, 0))],
        out_specs=[
            pl.BlockSpec((1, 1, 128), lambda b: (b, 0, 0)),
            pl.BlockSpec((1, 1, 128), lambda b: (b, 0, 0)),
        ],
        out_shape=[
            jax.ShapeDtypeStruct((bsz, 1, 128), jnp.float32),
            jax.ShapeDtypeStruct((bsz, 1, 128), jnp.int32),
        ],
    )(w_all)


def _agg_kernel(dhi_ref, dlo_ref, w_ref, vv_ref, o_ref, *, ktop, tl, cdim):
    b = pl.program_id(0)
    lt = pl.program_id(1)
    l0 = lt * tl
    win = tl + 8

    def body(it, acc):
        dhi = dhi_ref[b, it]
        dlo = dlo_ref[b, it]
        wgt = w_ref[b, it]
        base = pl.multiple_of(l0 + dhi * 8, 8)
        chunk = vv_ref[0, pl.ds(base, win), :]
        rolled = pltpu.roll(chunk, (win - dlo) % win, axis=0)
        return acc + wgt * rolled[:tl, :]

    acc0 = jnp.zeros((tl, cdim), jnp.float32)
    o_ref[0] = jax.lax.fori_loop(0, ktop, body, acc0)


def _delay_aggregate(vv, dhi, dlo, weights, ktop):
    bsz, two_l, cdim = vv.shape
    l = two_l // 2
    nl = l // _TL
    kern = lambda hr, lr, wr, vr, orf: _agg_kernel(
        hr, lr, wr, vr, orf, ktop=ktop, tl=_TL, cdim=cdim
    )
    return pl.pallas_call(
        kern,
        grid=(bsz, nl),
        in_specs=[
            pl.BlockSpec(memory_space=pltpu.SMEM),
            pl.BlockSpec(memory_space=pltpu.SMEM),
            pl.BlockSpec(memory_space=pltpu.SMEM),
            pl.BlockSpec((1, two_l, cdim), lambda b, t: (b, 0, 0)),
        ],
        out_specs=pl.BlockSpec((1, _TL, cdim), lambda b, t: (b, t, 0)),
        out_shape=jax.ShapeDtypeStruct((bsz, l, cdim), jnp.float32),
    )(dhi, dlo, weights, vv)


def kernel(x, Wq, bq, Wk, bk, Wv, bv, Wp, bp):
    bsz, t, c = x.shape
    h = _HEADS
    e = c // h
    l = t
    ktop = int(_LOGK_FACTOR * math.log(l))

    x2 = x.reshape(bsz * t, c)
    w_all = jnp.concatenate([Wq.T, Wk.T, Wv.T], axis=1)
    b_all = jnp.concatenate([bq, bk, bv]).reshape(1, 3 * c)
    qkv = _matmul_bias(x2, w_all, b_all)
    q = qkv[:, :c].reshape(bsz, t, c)
    k = qkv[:, c:2 * c].reshape(bsz, t, c)
    v = qkv[:, 2 * c:].reshape(bsz, t, c)

    corr_parts = _corr_partials(q, k)
    weights, delays = _topk_softmax(corr_parts, l, c, ktop)
    weights = weights.reshape(bsz, 128)
    delays = delays.reshape(bsz, 128)

    vv = jnp.concatenate([v, v], axis=1)
    nat = _delay_aggregate(vv, delays // 8, delays % 8, weights, ktop)

    scr = nat.reshape(bsz, l, h, e).transpose(0, 2, 1, 3).reshape(bsz * t, c)
    out = _matmul_bias(scr, Wp.T, bp.reshape(1, c))
    return out.reshape(bsz, t, c)


# speedup vs baseline: 11.9821x; 1.0502x over previous
import math

import jax
import jax.numpy as jnp
from jax.experimental import pallas as pl
from jax.experimental.pallas import tpu as pltpu

_HEADS = 12
_LOGK_FACTOR = 5
_TB = 512
_TM = 512
_TL = 512


def _mm_bias_kernel(x_ref, w_ref, b_ref, o_ref):
    xb = x_ref[...].astype(jnp.bfloat16)
    wb = w_ref[...].astype(jnp.bfloat16)
    o_ref[...] = (
        jnp.dot(xb, wb, preferred_element_type=jnp.float32) + b_ref[...]
    )


def _matmul_bias(x2d, w, bias):
    m, kdim = x2d.shape
    n = w.shape[1]
    grid = (m // _TM,)
    return pl.pallas_call(
        _mm_bias_kernel,
        grid=grid,
        in_specs=[
            pl.BlockSpec((_TM, kdim), lambda i: (i, 0)),
            pl.BlockSpec((kdim, n), lambda i: (0, 0)),
            pl.BlockSpec((1, n), lambda i: (0, 0)),
        ],
        out_specs=pl.BlockSpec((_TM, n), lambda i: (i, 0)),
        out_shape=jax.ShapeDtypeStruct((m, n), jnp.float32),
    )(x2d, w, bias)


def _corr_kernel(q_ref, k_ref, o_ref, *, nt, tb):
    d = pl.program_id(1)
    i = pl.program_id(2)

    @pl.when((d == 0) & (i == 0))
    def _():
        o_ref[...] = jnp.zeros_like(o_ref)

    qf, kf = q_ref[0], k_ref[0]
    qh = qf.astype(jnp.bfloat16)
    ql = (qf - qh.astype(jnp.float32)).astype(jnp.bfloat16)
    kh = kf.astype(jnp.bfloat16)
    kl = (kf - kh.astype(jnp.float32)).astype(jnp.bfloat16)
    dims = (((1,), (1,)), ((), ()))
    dotg = lambda a, b: jax.lax.dot_general(
        a, b, dims, preferred_element_type=jnp.float32)
    s = dotg(qh, kh) + dotg(qh, kl) + dotg(ql, kh)

    a_i = jax.lax.broadcasted_iota(jnp.int32, (tb, tb), 0)
    c_i = jax.lax.broadcasted_iota(jnp.int32, (tb, tb), 1)

    rolled = pltpu.roll(s, 0, axis=1, stride=1, stride_axis=0)
    nowrap = c_i >= a_i
    colsum1 = jnp.sum(jnp.where(nowrap, rolled, 0.0), axis=0, keepdims=True)
    colsum2 = jnp.sum(jnp.where(nowrap, 0.0, rolled), axis=0, keepdims=True)
    contrib = jnp.concatenate([colsum1, colsum2], axis=1)

    cur = o_ref[0, pl.ds(d, 1), :]
    o_ref[0, pl.ds(d, 1), :] = cur + contrib


def _corr_partials(q, k):
    bsz, l, c = q.shape
    nt = l // _TB
    kern = lambda qr, kr, orf: _corr_kernel(qr, kr, orf, nt=nt, tb=_TB)
    return pl.pallas_call(
        kern,
        grid=(bsz, nt, nt),
        in_specs=[
            pl.BlockSpec((1, _TB, c), lambda b, d, i: (b, i, 0)),
            pl.BlockSpec((1, _TB, c), lambda b, d, i: (b, (d - i) % nt, 0)),
        ],
        out_specs=pl.BlockSpec((1, nt, 2 * _TB), lambda b, d, i: (b, 0, 0)),
        out_shape=jax.ShapeDtypeStruct((bsz, nt, 2 * _TB), jnp.float32),
    )(q, k)


def _topk_kernel(w_ref, wout_ref, dout_ref, *, nt, tb, l, ktop, cdim):
    w = w_ref[0]
    m_b = jnp.concatenate([w[-1:, tb:], w[:-1, tb:]], axis=0)
    m0 = (w[:, :tb] + m_b) * (1.0 / cdim)
    dd = jax.lax.broadcasted_iota(jnp.int32, (nt, tb), 0)
    ss = jax.lax.broadcasted_iota(jnp.int32, (nt, tb), 1)
    tau = (l - 1 - (dd * tb + ss)) % l

    lane = jax.lax.broadcasted_iota(jnp.int32, (1, 128), 1)
    neg_inf = jnp.float32(-jnp.inf)

    def body(it, carry):
        m_cur, vals, idxs = carry
        cur = jnp.max(m_cur)
        idx = jnp.min(jnp.where(m_cur == cur, tau, l))
        vals = jnp.where(lane == it, cur, vals)
        idxs = jnp.where(lane == it, idx, idxs)
        m_cur = jnp.where((m_cur == cur) & (tau == idx), neg_inf, m_cur)
        return m_cur, vals, idxs

    vals0 = jnp.full((1, 128), neg_inf, jnp.float32)
    idxs0 = jnp.zeros((1, 128), jnp.int32)
    _, vals, idxs = jax.lax.fori_loop(0, ktop, body, (m0, vals0, idxs0))

    active = lane < ktop
    mx = jnp.max(vals)
    e = jnp.where(active, jnp.exp(vals - mx), 0.0)
    wout_ref[0] = e / jnp.sum(e)
    dout_ref[0] = idxs


def _topk_softmax(w_all, l, cdim, ktop):
    bsz, nt, two_tb = w_all.shape
    tb = two_tb // 2
    kern = lambda wr, wo, do: _topk_kernel(
        wr, wo, do, nt=nt, tb=tb, l=l, ktop=ktop, cdim=cdim
    )
    return pl.pallas_call(
        kern,
        grid=(bsz,),
        in_specs=[pl.BlockSpec((1, nt, two_tb), lambda b: (b, 0, 0))],
        out_specs=[
            pl.BlockSpec((1, 1, 128), lambda b: (b, 0, 0)),
            pl.BlockSpec((1, 1, 128), lambda b: (b, 0, 0)),
        ],
        out_shape=[
            jax.ShapeDtypeStruct((bsz, 1, 128), jnp.float32),
            jax.ShapeDtypeStruct((bsz, 1, 128), jnp.int32),
        ],
    )(w_all)


def _agg_kernel(dhi_ref, dlo_ref, w_ref, vv_ref, o_ref, *, ktop, tl, cdim):
    b = pl.program_id(0)
    lt = pl.program_id(1)
    l0 = lt * tl
    win = tl + 8

    def body(it, acc):
        dhi = dhi_ref[b, it]
        dlo = dlo_ref[b, it]
        wgt = w_ref[b, it]
        base = pl.multiple_of(l0 + dhi * 8, 8)
        chunk = vv_ref[0, pl.ds(base, win), :]
        rolled = pltpu.roll(chunk, (win - dlo) % win, axis=0)
        return acc + wgt * rolled[:tl, :]

    acc0 = jnp.zeros((tl, cdim), jnp.float32)
    o_ref[0] = jax.lax.fori_loop(0, ktop, body, acc0)


def _delay_aggregate(vv, dhi, dlo, weights, ktop):
    bsz, two_l, cdim = vv.shape
    l = two_l // 2
    nl = l // _TL
    kern = lambda hr, lr, wr, vr, orf: _agg_kernel(
        hr, lr, wr, vr, orf, ktop=ktop, tl=_TL, cdim=cdim
    )
    return pl.pallas_call(
        kern,
        grid=(bsz, nl),
        in_specs=[
            pl.BlockSpec(memory_space=pltpu.SMEM),
            pl.BlockSpec(memory_space=pltpu.SMEM),
            pl.BlockSpec(memory_space=pltpu.SMEM),
            pl.BlockSpec((1, two_l, cdim), lambda b, t: (b, 0, 0)),
        ],
        out_specs=pl.BlockSpec((1, _TL, cdim), lambda b, t: (b, t, 0)),
        out_shape=jax.ShapeDtypeStruct((bsz, l, cdim), jnp.float32),
    )(dhi, dlo, weights, vv)


def kernel(x, Wq, bq, Wk, bk, Wv, bv, Wp, bp):
    bsz, t, c = x.shape
    h = _HEADS
    e = c // h
    l = t
    ktop = int(_LOGK_FACTOR * math.log(l))

    x2 = x.reshape(bsz * t, c)
    w_all = jnp.concatenate([Wq.T, Wk.T, Wv.T], axis=1)
    b_all = jnp.concatenate([bq, bk, bv]).reshape(1, 3 * c)
    qkv = _matmul_bias(x2, w_all, b_all)
    q = qkv[:, :c].reshape(bsz, t, c)
    k = qkv[:, c:2 * c].reshape(bsz, t, c)
    v = qkv[:, 2 * c:].reshape(bsz, t, c)

    corr_parts = _corr_partials(jnp.flip(q, axis=1), k)
    weights, delays = _topk_softmax(corr_parts, l, c, ktop)
    weights = weights.reshape(bsz, 128)
    delays = delays.reshape(bsz, 128)

    vv = jnp.concatenate([v, v], axis=1)
    nat = _delay_aggregate(vv, delays // 8, delays % 8, weights, ktop)

    scr = nat.reshape(bsz, l, h, e).transpose(0, 2, 1, 3).reshape(bsz * t, c)
    out = _matmul_bias(scr, Wp.T, bp.reshape(1, c))
    return out.reshape(bsz, t, c)


# speedup vs baseline: 11.9829x; 1.0001x over previous
import math

import jax
import jax.numpy as jnp
from jax.experimental import pallas as pl
from jax.experimental.pallas import tpu as pltpu

_HEADS = 12
_LOGK_FACTOR = 5
_TB = 512
_TM = 512
_TL = 512


def _mm_bias_kernel(x_ref, w_ref, b_ref, o_ref):
    xb = x_ref[...].astype(jnp.bfloat16)
    wb = w_ref[...].astype(jnp.bfloat16)
    o_ref[...] = (
        jnp.dot(xb, wb, preferred_element_type=jnp.float32) + b_ref[...]
    )


def _matmul_bias(x2d, w, bias):
    m, kdim = x2d.shape
    n = w.shape[1]
    grid = (m // _TM,)
    return pl.pallas_call(
        _mm_bias_kernel,
        grid=grid,
        in_specs=[
            pl.BlockSpec((_TM, kdim), lambda i: (i, 0)),
            pl.BlockSpec((kdim, n), lambda i: (0, 0)),
            pl.BlockSpec((1, n), lambda i: (0, 0)),
        ],
        out_specs=pl.BlockSpec((_TM, n), lambda i: (i, 0)),
        out_shape=jax.ShapeDtypeStruct((m, n), jnp.float32),
    )(x2d, w, bias)


def _corr_kernel(q_ref, k_ref, o_ref, *, tb):
    d = pl.program_id(1)
    i = pl.program_id(2)

    @pl.when((d == 0) & (i == 0))
    def _():
        o_ref[...] = jnp.zeros_like(o_ref)

    qf, kf = q_ref[0], k_ref[0]
    qh = qf.astype(jnp.bfloat16)
    ql = (qf - qh.astype(jnp.float32)).astype(jnp.bfloat16)
    kh = kf.astype(jnp.bfloat16)
    kl = (kf - kh.astype(jnp.float32)).astype(jnp.bfloat16)
    dims = (((1,), (1,)), ((), ()))
    dotg = lambda a, b: jax.lax.dot_general(
        a, b, dims, preferred_element_type=jnp.float32)
    s = dotg(qh, kh) + dotg(qh, kl) + dotg(ql, kh)

    a_i = jax.lax.broadcasted_iota(jnp.int32, (tb, tb), 0)
    c_i = jax.lax.broadcasted_iota(jnp.int32, (tb, tb), 1)

    rolled = pltpu.roll(s, 0, axis=1, stride=1, stride_axis=0)
    nowrap = c_i >= a_i
    colsum1 = jnp.sum(jnp.where(nowrap, rolled, 0.0), axis=0, keepdims=True)
    colsum2 = jnp.sum(jnp.where(nowrap, 0.0, rolled), axis=0, keepdims=True)
    contrib = jnp.concatenate([colsum1, colsum2], axis=1)

    cur = o_ref[0, pl.ds(d, 1), :]
    o_ref[0, pl.ds(d, 1), :] = cur + contrib


def _corr_partials(q, k):
    bsz, l, c = q.shape
    nt = l // _TB
    kern = lambda qr, kr, orf: _corr_kernel(qr, kr, orf, tb=_TB)
    return pl.pallas_call(
        kern,
        grid=(bsz, nt, nt),
        in_specs=[
            pl.BlockSpec((1, _TB, c), lambda b, d, i: (b, i, 0)),
            pl.BlockSpec((1, _TB, c), lambda b, d, i: (b, (d - i) % nt, 0)),
        ],
        out_specs=pl.BlockSpec((1, nt, 2 * _TB), lambda b, d, i: (b, 0, 0)),
        out_shape=jax.ShapeDtypeStruct((bsz, nt, 2 * _TB), jnp.float32),
    )(q, k)


def _topk_kernel(w_ref, wout_ref, dout_ref, *, nt, tb, l, ktop, cdim):
    w = w_ref[0]
    m_b = jnp.concatenate([w[-1:, tb:], w[:-1, tb:]], axis=0)
    m0 = (w[:, :tb] + m_b) * (1.0 / cdim)
    dd = jax.lax.broadcasted_iota(jnp.int32, (nt, tb), 0)
    ss = jax.lax.broadcasted_iota(jnp.int32, (nt, tb), 1)
    tau = (l - 1 - (dd * tb + ss)) % l

    lane = jax.lax.broadcasted_iota(jnp.int32, (1, 128), 1)
    neg_inf = jnp.float32(-jnp.inf)

    def body(it, carry):
        m_cur, vals, idxs = carry
        cur = jnp.max(m_cur)
        idx = jnp.min(jnp.where(m_cur == cur, tau, l))
        vals = jnp.where(lane == it, cur, vals)
        idxs = jnp.where(lane == it, idx, idxs)
        m_cur = jnp.where((m_cur == cur) & (tau == idx), neg_inf, m_cur)
        return m_cur, vals, idxs

    vals0 = jnp.full((1, 128), neg_inf, jnp.float32)
    idxs0 = jnp.zeros((1, 128), jnp.int32)
    _, vals, idxs = jax.lax.fori_loop(0, ktop, body, (m0, vals0, idxs0))

    active = lane < ktop
    mx = jnp.max(vals)
    e = jnp.where(active, jnp.exp(vals - mx), 0.0)
    wout_ref[0] = e / jnp.sum(e)
    dout_ref[0] = idxs


def _topk_softmax(w_all, l, cdim, ktop):
    bsz, nt, two_tb = w_all.shape
    tb = two_tb // 2
    kern = lambda wr, wo, do: _topk_kernel(
        wr, wo, do, nt=nt, tb=tb, l=l, ktop=ktop, cdim=cdim
    )
    return pl.pallas_call(
        kern,
        grid=(bsz,),
        in_specs=[pl.BlockSpec((1, nt, two_tb), lambda b: (b, 0, 0))],
        out_specs=[
            pl.BlockSpec((1, 1, 128), lambda b: (b, 0, 0)),
            pl.BlockSpec((1, 1, 128), lambda b: (b, 0, 0)),
        ],
        out_shape=[
            jax.ShapeDtypeStruct((bsz, 1, 128), jnp.float32),
            jax.ShapeDtypeStruct((bsz, 1, 128), jnp.int32),
        ],
    )(w_all)


def _agg_kernel(dhi_ref, dlo_ref, w_ref, vv_ref, o_ref, *, ktop, tl, cdim):
    b = pl.program_id(0)
    lt = pl.program_id(1)
    l0 = lt * tl
    win = tl + 8

    def body(it, acc):
        dhi = dhi_ref[b, it]
        dlo = dlo_ref[b, it]
        wgt = w_ref[b, it]
        base = pl.multiple_of(l0 + dhi * 8, 8)
        chunk = vv_ref[0, pl.ds(base, win), :]
        rolled = pltpu.roll(chunk, (win - dlo) % win, axis=0)
        return acc + wgt * rolled[:tl, :]

    acc0 = jnp.zeros((tl, cdim), jnp.float32)
    o_ref[0] = jax.lax.fori_loop(0, ktop, body, acc0)


def _delay_aggregate(vv, dhi, dlo, weights, ktop):
    bsz, two_l, cdim = vv.shape
    l = two_l // 2
    nl = l // _TL
    kern = lambda hr, lr, wr, vr, orf: _agg_kernel(
        hr, lr, wr, vr, orf, ktop=ktop, tl=_TL, cdim=cdim
    )
    return pl.pallas_call(
        kern,
        grid=(bsz, nl),
        in_specs=[
            pl.BlockSpec(memory_space=pltpu.SMEM),
            pl.BlockSpec(memory_space=pltpu.SMEM),
            pl.BlockSpec(memory_space=pltpu.SMEM),
            pl.BlockSpec((1, two_l, cdim), lambda b, t: (b, 0, 0)),
        ],
        out_specs=pl.BlockSpec((1, _TL, cdim), lambda b, t: (b, t, 0)),
        out_shape=jax.ShapeDtypeStruct((bsz, l, cdim), jnp.float32),
    )(dhi, dlo, weights, vv)


def kernel(x, Wq, bq, Wk, bk, Wv, bv, Wp, bp):
    bsz, t, c = x.shape
    h = _HEADS
    e = c // h
    l = t
    ktop = int(_LOGK_FACTOR * math.log(l))

    x2 = x.reshape(bsz * t, c)
    w_all = jnp.concatenate([Wq.T, Wk.T, Wv.T], axis=1)
    b_all = jnp.concatenate([bq, bk, bv]).reshape(1, 3 * c)
    qkv = _matmul_bias(x2, w_all, b_all)
    q = qkv[:, :c].reshape(bsz, t, c)
    k = qkv[:, c:2 * c].reshape(bsz, t, c)
    v = qkv[:, 2 * c:].reshape(bsz, t, c)

    corr_parts = _corr_partials(jnp.flip(q, axis=1), k)
    weights, delays = _topk_softmax(corr_parts, l, c, ktop)
    weights = weights.reshape(bsz, 128)
    delays = delays.reshape(bsz, 128)

    vv = jnp.concatenate([v, v], axis=1)
    nat = _delay_aggregate(vv, delays // 8, delays % 8, weights, ktop)

    scr = nat.reshape(bsz, l, h, e).transpose(0, 2, 1, 3).reshape(bsz * t, c)
    out = _matmul_bias(scr, Wp.T, bp.reshape(1, c))
    return out.reshape(bsz, t, c)
